# baseline (speedup 1.0000x reference)
import functools

import jax
import jax.numpy as jnp
from jax import lax
from jax.experimental import pallas as pl
from jax.experimental.pallas import tpu as pltpu
from jax.experimental.pallas import tpu_sc as plsc

N_NODES = 10000
C = 128
N_EDGES = 320000

NC = 2
NS = 16
NW = NC * NS
E_PER_TILE = N_EDGES // NW
CHUNK = 125
NCHUNK = E_PER_TILE // CHUNK
N_PAD = 10240
ROWS_PER_TILE = N_PAD // NS
_TAIL_ROWS = N_NODES - (NS - 1) * ROWS_PER_TILE

_sc_mesh = plsc.VectorSubcoreMesh(
    core_axis_name="c", subcore_axis_name="s", num_cores=NC, num_subcores=NS
)


def _wid():
    return lax.axis_index("c") * NS + lax.axis_index("s")


@functools.partial(
    pl.kernel,
    out_type=jax.ShapeDtypeStruct((NC, N_PAD), jnp.float32),
    mesh=_sc_mesh,
    scratch_types=[
        pltpu.VMEM((NCHUNK, CHUNK), jnp.int32),
        pltpu.VMEM((CHUNK,), jnp.float32),
        pltpu.SemaphoreType.DMA,
        pltpu.VMEM_SHARED((N_PAD,), jnp.float32),
    ],
)
def _sc_degree(dst_hbm, zeros_hbm, ones_hbm, degp_hbm, idx_v, ones_v, ssem,
               acc):
    cid = lax.axis_index("c")
    sid = lax.axis_index("s")
    row0 = sid * ROWS_PER_TILE
    pltpu.sync_copy(zeros_hbm.at[pl.ds(row0, ROWS_PER_TILE)],
                    acc.at[pl.ds(row0, ROWS_PER_TILE)])
    pltpu.sync_copy(ones_hbm, ones_v)
    pltpu.sync_copy(dst_hbm.at[_wid()], idx_v)
    plsc.subcore_barrier()

    @pl.loop(0, NCHUNK // 8)
    def _scatter(gq):
        for b in range(8):
            pltpu.async_copy(ones_v, acc.at[idx_v.at[gq * 8 + b]], ssem,
                             add=True)
        for b in range(8):
            pltpu.make_async_copy(ones_v, acc.at[idx_v.at[0]], ssem).wait()

    plsc.subcore_barrier()
    pltpu.sync_copy(acc.at[pl.ds(row0, ROWS_PER_TILE)],
                    degp_hbm.at[cid, pl.ds(row0, ROWS_PER_TILE)])


NBUF = 2
HALF = NCHUNK // 2
NGROUP = HALF // NBUF


@functools.partial(
    pl.kernel,
    out_type=jax.ShapeDtypeStruct((NC, N_PAD, C), jnp.float32),
    mesh=_sc_mesh,
    scratch_types=[
        pltpu.VMEM((HALF, CHUNK), jnp.int32),
        pltpu.VMEM((HALF, CHUNK), jnp.int32),
        [pltpu.VMEM((CHUNK, C), jnp.float32) for _ in range(NBUF)],
        [pltpu.SemaphoreType.DMA for _ in range(NBUF)],
        pltpu.VMEM_SHARED((N_PAD, C), jnp.float32),
    ],
)
def _sc_aggregate(g_hbm, src_hbm, dst_hbm, aggp_hbm,
                  idxs_v, idxd_v, bufs, gsems, acc):
    cid = lax.axis_index("c")
    sid = lax.axis_index("s")
    row0 = sid * ROWS_PER_TILE

    @pl.when(sid < NS - 1)
    def _full():
        pltpu.sync_copy(g_hbm.at[pl.ds(row0, ROWS_PER_TILE)],
                        acc.at[pl.ds(row0, ROWS_PER_TILE)])

    @pl.when(sid == NS - 1)
    def _tail():
        pltpu.sync_copy(g_hbm.at[pl.ds(row0, _TAIL_ROWS)],
                        acc.at[pl.ds(row0, _TAIL_ROWS)])

    plsc.subcore_barrier()

    for h in range(2):
        pltpu.sync_copy(src_hbm.at[_wid(), pl.ds(h * HALF, HALF)], idxs_v)
        pltpu.sync_copy(dst_hbm.at[_wid(), pl.ds(h * HALF, HALF)], idxd_v)
        for b in range(NBUF):
            pltpu.async_copy(g_hbm.at[idxs_v.at[b]], bufs[b], gsems[b])

        @pl.loop(0, NGROUP)
        def _grp(g):
            for b in range(NBUF):
                j = g * NBUF + b
                pltpu.make_async_copy(g_hbm.at[idxs_v.at[0]], bufs[b],
                                      gsems[b]).wait()
                pltpu.sync_copy(bufs[b], acc.at[idxd_v.at[j]], add=True)

                @pl.when(g < NGROUP - 1)
                def _prefetch():
                    pltpu.async_copy(g_hbm.at[idxs_v.at[j + NBUF]],
                                     bufs[b], gsems[b])

    plsc.subcore_barrier()
    pltpu.sync_copy(acc.at[pl.ds(row0, ROWS_PER_TILE)],
                    aggp_hbm.at[cid, pl.ds(row0, ROWS_PER_TILE)])


_RB = 2000
_GRID = N_NODES // _RB


def _tc_phase1_body(degp_ref, x_ref, w1_ref, dinv_ref, g1_ref):
    deg = degp_ref[0] + degp_ref[1] + 1.0
    dinv = lax.rsqrt(deg)
    dinv_ref[...] = dinv
    g1_ref[...] = dinv * jnp.dot(x_ref[...], w1_ref[...],
                                 preferred_element_type=jnp.float32)


def _tc_phase2_body(aggp_ref, g_ref, dinv_ref, b1_ref, w2_ref, g2_ref):
    s = aggp_ref[0] + aggp_ref[1] - g_ref[...]
    h = dinv_ref[...] * s + b1_ref[...]
    r = jnp.maximum(h, 0.0)
    g2_ref[...] = dinv_ref[...] * jnp.dot(r, w2_ref[...],
                                          preferred_element_type=jnp.float32)


def _tc_phase3_body(aggp_ref, g_ref, dinv_ref, b2_ref, out_ref):
    s = aggp_ref[0] + aggp_ref[1] - g_ref[...]
    out_ref[...] = dinv_ref[...] * s + b2_ref[...]


def _rows_spec(width):
    return pl.BlockSpec((_RB, width), lambda i: (i, 0))


def _aggp_spec(width):
    return pl.BlockSpec((2, _RB, width), lambda i: (0, i, 0))


def _full_spec(shape):
    return pl.BlockSpec(shape, lambda i: tuple(0 for _ in shape))


def kernel(x, edge_index, W1, b1, W2, b2):
    ei = edge_index.astype(jnp.int32)
    src3 = ei[0].reshape(NW, NCHUNK, CHUNK)
    dst3 = ei[1].reshape(NW, NCHUNK, CHUNK)
    zeros1 = jnp.zeros((N_PAD,), jnp.float32)
    ones_chunk = jnp.ones((CHUNK,), jnp.float32)
    b1r = b1.reshape(1, C)
    b2r = b2.reshape(1, C)

    degp = _sc_degree(dst3, zeros1, ones_chunk).reshape(NC, N_PAD, 1)

    dinv, g1 = pl.pallas_call(
        _tc_phase1_body,
        grid=(_GRID,),
        in_specs=[_aggp_spec(1), _rows_spec(C), _full_spec((C, C))],
        out_specs=[_rows_spec(1), _rows_spec(C)],
        out_shape=[
            jax.ShapeDtypeStruct((N_NODES, 1), jnp.float32),
            jax.ShapeDtypeStruct((N_NODES, C), jnp.float32),
        ],
    )(degp, x, W1)

    aggp1 = _sc_aggregate(g1, src3, dst3)

    g2 = pl.pallas_call(
        _tc_phase2_body,
        grid=(_GRID,),
        in_specs=[_aggp_spec(C), _rows_spec(C), _rows_spec(1),
                  _full_spec((1, C)), _full_spec((C, C))],
        out_specs=_rows_spec(C),
        out_shape=jax.ShapeDtypeStruct((N_NODES, C), jnp.float32),
    )(aggp1, g1, dinv, b1r, W2)

    aggp2 = _sc_aggregate(g2, src3, dst3)

    out = pl.pallas_call(
        _tc_phase3_body,
        grid=(_GRID,),
        in_specs=[_aggp_spec(C), _rows_spec(C), _rows_spec(1),
                  _full_spec((1, C))],
        out_specs=_rows_spec(C),
        out_shape=jax.ShapeDtypeStruct((N_NODES, C), jnp.float32),
    )(aggp2, g2, dinv, b2r)

    return out


# speedup vs baseline: 1.0014x; 1.0014x over previous
import functools

import jax
import jax.numpy as jnp
from jax import lax
from jax.experimental import pallas as pl
from jax.experimental.pallas import tpu as pltpu
from jax.experimental.pallas import tpu_sc as plsc

N_NODES = 10000
C = 128
N_EDGES = 320000

NC = 2
NS = 16
NW = NC * NS
E_PER_TILE = N_EDGES // NW
CHUNK = 125
NCHUNK = E_PER_TILE // CHUNK
N_PAD = 10240
ROWS_PER_TILE = N_PAD // NS
_TAIL_ROWS = N_NODES - (NS - 1) * ROWS_PER_TILE

_sc_mesh = plsc.VectorSubcoreMesh(
    core_axis_name="c", subcore_axis_name="s", num_cores=NC, num_subcores=NS
)


def _wid():
    return lax.axis_index("c") * NS + lax.axis_index("s")


@functools.partial(
    pl.kernel,
    out_type=jax.ShapeDtypeStruct((NC, N_PAD), jnp.float32),
    mesh=_sc_mesh,
    scratch_types=[
        pltpu.VMEM((NCHUNK, CHUNK), jnp.int32),
        pltpu.VMEM((CHUNK,), jnp.float32),
        pltpu.SemaphoreType.DMA,
        pltpu.VMEM_SHARED((N_PAD,), jnp.float32),
    ],
)
def _sc_degree(dst_hbm, zeros_hbm, ones_hbm, degp_hbm, idx_v, ones_v, ssem,
               acc):
    cid = lax.axis_index("c")
    sid = lax.axis_index("s")
    row0 = sid * ROWS_PER_TILE
    pltpu.sync_copy(zeros_hbm.at[pl.ds(row0, ROWS_PER_TILE)],
                    acc.at[pl.ds(row0, ROWS_PER_TILE)])
    pltpu.sync_copy(ones_hbm, ones_v)
    pltpu.sync_copy(dst_hbm.at[_wid()], idx_v)
    plsc.subcore_barrier()

    @pl.loop(0, NCHUNK // 16)
    def _scatter(gq):
        for b in range(16):
            pltpu.async_copy(ones_v, acc.at[idx_v.at[gq * 16 + b]], ssem,
                             add=True)
        for b in range(16):
            pltpu.make_async_copy(ones_v, acc.at[idx_v.at[0]], ssem).wait()

    plsc.subcore_barrier()
    pltpu.sync_copy(acc.at[pl.ds(row0, ROWS_PER_TILE)],
                    degp_hbm.at[cid, pl.ds(row0, ROWS_PER_TILE)])


NBUF = 2
HALF = NCHUNK // 2
NGROUP = HALF // NBUF


@functools.partial(
    pl.kernel,
    out_type=jax.ShapeDtypeStruct((NC, N_PAD, C), jnp.float32),
    mesh=_sc_mesh,
    scratch_types=[
        pltpu.VMEM((HALF, CHUNK), jnp.int32),
        pltpu.VMEM((HALF, CHUNK), jnp.int32),
        [pltpu.VMEM((CHUNK, C), jnp.float32) for _ in range(NBUF)],
        [pltpu.SemaphoreType.DMA for _ in range(NBUF)],
        pltpu.VMEM_SHARED((N_PAD, C), jnp.float32),
    ],
)
def _sc_aggregate(g_hbm, src_hbm, dst_hbm, aggp_hbm,
                  idxs_v, idxd_v, bufs, gsems, acc):
    cid = lax.axis_index("c")
    sid = lax.axis_index("s")
    row0 = sid * ROWS_PER_TILE

    @pl.when(sid < NS - 1)
    def _full():
        pltpu.sync_copy(g_hbm.at[pl.ds(row0, ROWS_PER_TILE)],
                        acc.at[pl.ds(row0, ROWS_PER_TILE)])

    @pl.when(sid == NS - 1)
    def _tail():
        pltpu.sync_copy(g_hbm.at[pl.ds(row0, _TAIL_ROWS)],
                        acc.at[pl.ds(row0, _TAIL_ROWS)])

    pltpu.sync_copy(src_hbm.at[_wid(), pl.ds(0, HALF)], idxs_v)
    pltpu.sync_copy(dst_hbm.at[_wid(), pl.ds(0, HALF)], idxd_v)
    for b in range(NBUF):
        pltpu.async_copy(g_hbm.at[idxs_v.at[b]], bufs[b], gsems[b])
    plsc.subcore_barrier()

    for h in range(2):
        if h > 0:
            pltpu.sync_copy(src_hbm.at[_wid(), pl.ds(h * HALF, HALF)], idxs_v)
            pltpu.sync_copy(dst_hbm.at[_wid(), pl.ds(h * HALF, HALF)], idxd_v)
            for b in range(NBUF):
                pltpu.async_copy(g_hbm.at[idxs_v.at[b]], bufs[b], gsems[b])

        @pl.loop(0, NGROUP)
        def _grp(g):
            for b in range(NBUF):
                j = g * NBUF + b
                pltpu.make_async_copy(g_hbm.at[idxs_v.at[0]], bufs[b],
                                      gsems[b]).wait()
                pltpu.sync_copy(bufs[b], acc.at[idxd_v.at[j]], add=True)

                @pl.when(g < NGROUP - 1)
                def _prefetch():
                    pltpu.async_copy(g_hbm.at[idxs_v.at[j + NBUF]],
                                     bufs[b], gsems[b])

    plsc.subcore_barrier()
    pltpu.sync_copy(acc.at[pl.ds(row0, ROWS_PER_TILE)],
                    aggp_hbm.at[cid, pl.ds(row0, ROWS_PER_TILE)])


_RB = 2000
_GRID = N_NODES // _RB


def _tc_phase1_body(degp_ref, x_ref, w1_ref, dinv_ref, g1_ref):
    deg = degp_ref[0] + degp_ref[1] + 1.0
    dinv = lax.rsqrt(deg)
    dinv_ref[...] = dinv
    g1_ref[...] = dinv * jnp.dot(x_ref[...], w1_ref[...],
                                 preferred_element_type=jnp.float32)


def _tc_phase2_body(aggp_ref, g_ref, dinv_ref, b1_ref, w2_ref, g2_ref):
    s = aggp_ref[0] + aggp_ref[1] - g_ref[...]
    h = dinv_ref[...] * s + b1_ref[...]
    r = jnp.maximum(h, 0.0)
    g2_ref[...] = dinv_ref[...] * jnp.dot(r, w2_ref[...],
                                          preferred_element_type=jnp.float32)


def _tc_phase3_body(aggp_ref, g_ref, dinv_ref, b2_ref, out_ref):
    s = aggp_ref[0] + aggp_ref[1] - g_ref[...]
    out_ref[...] = dinv_ref[...] * s + b2_ref[...]


def _rows_spec(width):
    return pl.BlockSpec((_RB, width), lambda i: (i, 0))


def _aggp_spec(width):
    return pl.BlockSpec((2, _RB, width), lambda i: (0, i, 0))


def _full_spec(shape):
    return pl.BlockSpec(shape, lambda i: tuple(0 for _ in shape))


def kernel(x, edge_index, W1, b1, W2, b2):
    ei = edge_index.astype(jnp.int32)
    src3 = ei[0].reshape(NW, NCHUNK, CHUNK)
    dst3 = ei[1].reshape(NW, NCHUNK, CHUNK)
    zeros1 = jnp.zeros((N_PAD,), jnp.float32)
    ones_chunk = jnp.ones((CHUNK,), jnp.float32)
    b1r = b1.reshape(1, C)
    b2r = b2.reshape(1, C)

    degp = _sc_degree(dst3, zeros1, ones_chunk).reshape(NC, N_PAD, 1)

    dinv, g1 = pl.pallas_call(
        _tc_phase1_body,
        grid=(_GRID,),
        in_specs=[_aggp_spec(1), _rows_spec(C), _full_spec((C, C))],
        out_specs=[_rows_spec(1), _rows_spec(C)],
        out_shape=[
            jax.ShapeDtypeStruct((N_NODES, 1), jnp.float32),
            jax.ShapeDtypeStruct((N_NODES, C), jnp.float32),
        ],
    )(degp, x, W1)

    aggp1 = _sc_aggregate(g1, src3, dst3)

    g2 = pl.pallas_call(
        _tc_phase2_body,
        grid=(_GRID,),
        in_specs=[_aggp_spec(C), _rows_spec(C), _rows_spec(1),
                  _full_spec((1, C)), _full_spec((C, C))],
        out_specs=_rows_spec(C),
        out_shape=jax.ShapeDtypeStruct((N_NODES, C), jnp.float32),
    )(aggp1, g1, dinv, b1r, W2)

    aggp2 = _sc_aggregate(g2, src3, dst3)

    out = pl.pallas_call(
        _tc_phase3_body,
        grid=(_GRID,),
        in_specs=[_aggp_spec(C), _rows_spec(C), _rows_spec(1),
                  _full_spec((1, C))],
        out_specs=_rows_spec(C),
        out_shape=jax.ShapeDtypeStruct((N_NODES, C), jnp.float32),
    )(aggp2, g2, dinv, b2r)

    return out
